# baseline (speedup 1.0000x reference)
import numpy as np
import jax
import jax.numpy as jnp
from jax.experimental import pallas as pl
from jax.experimental.pallas import tpu as pltpu

_THETA = np.float32(np.log(2 / 1 + 1))

_N = 10000
_D = 128
_BM = 200
_NBUF = 4
_STEPS = _N // _BM


def _copy(adj_hbm_ref, buf_ref, sem_ref, panel, slot):
    return pltpu.make_async_copy(
        adj_hbm_ref.at[pl.ds(panel * _BM, _BM), :],
        buf_ref.at[slot],
        sem_ref.at[slot],
    )


def _body(a_ref, adj_hbm_ref, x_ref, rx_ref, rl_ref, w_ref, out_ref, buf_ref, xbf_ref, sem_ref):
    i = pl.program_id(0)

    @pl.when(i == 0)
    def _():
        xbf_ref[...] = x_ref[...].astype(jnp.bfloat16)
        for p in range(_NBUF):
            _copy(adj_hbm_ref, buf_ref, sem_ref, p, p).start()

    slot = jax.lax.rem(i, _NBUF)
    _copy(adj_hbm_ref, buf_ref, sem_ref, i, slot).wait()

    al = a_ref[0, 0]
    c1 = (1.0 - _THETA) * (1.0 - al)
    c2 = (1.0 - _THETA) * al
    row = jax.lax.broadcasted_iota(jnp.int32, (_D, _D), 0)
    col = jax.lax.broadcasted_iota(jnp.int32, (_D, _D), 1)
    eye = (row == col).astype(jnp.float32)
    a_mat = _THETA * w_ref[:_D, :] + c1 * eye
    b_mat = _THETA * w_ref[_D:, :] + c2 * eye
    hi = jnp.dot(
        buf_ref[slot].astype(jnp.bfloat16),
        xbf_ref[...],
        preferred_element_type=jnp.float32,
    )
    out_ref[...] = (
        jnp.dot(hi, a_mat, preferred_element_type=jnp.float32)
        + jnp.dot(rx_ref[...], b_mat[: _D // 2, :], preferred_element_type=jnp.float32)
        + jnp.dot(rl_ref[...], b_mat[_D // 2 :, :], preferred_element_type=jnp.float32)
    )

    @pl.when(i + _NBUF < _STEPS)
    def _():
        _copy(adj_hbm_ref, buf_ref, sem_ref, i + _NBUF, slot).start()


def kernel(input, adj, Rxyz, Rlamda, alpha, weight, t, l):
    del t, l
    alpha2d = jnp.reshape(alpha.astype(jnp.float32), (1, 1))

    out = pl.pallas_call(
        _body,
        grid=(_STEPS,),
        in_specs=[
            pl.BlockSpec(memory_space=pltpu.SMEM),
            pl.BlockSpec(memory_space=pl.ANY),
            pl.BlockSpec((_N, _D), lambda i: (0, 0)),
            pl.BlockSpec((_BM, _D // 2), lambda i: (i, 0)),
            pl.BlockSpec((_BM, _D // 2), lambda i: (i, 0)),
            pl.BlockSpec((2 * _D, _D), lambda i: (0, 0)),
        ],
        out_specs=pl.BlockSpec((_BM, _D), lambda i: (i, 0)),
        out_shape=jax.ShapeDtypeStruct((_N, _D), jnp.float32),
        scratch_shapes=[
            pltpu.VMEM((_NBUF, _BM, _N), jnp.float32),
            pltpu.VMEM((_N, _D), jnp.bfloat16),
            pltpu.SemaphoreType.DMA((_NBUF,)),
        ],
        compiler_params=pltpu.CompilerParams(
            dimension_semantics=("arbitrary",),
        ),
    )(alpha2d, adj, input, Rxyz, Rlamda, weight)
    return out


# speedup vs baseline: 1.0362x; 1.0362x over previous
import numpy as np
import jax
import jax.numpy as jnp
from jax.experimental import pallas as pl
from jax.experimental.pallas import tpu as pltpu

_THETA = np.float32(np.log(2 / 1 + 1))

_N = 10000
_D = 128
_BM = 400
_NS = 2
_BH = _BM // _NS


def _body(a_ref, adj0_ref, adj1_ref, x_ref, rx_ref, rl_ref, w_ref, out_ref, xbf_ref):
    i = pl.program_id(0)

    @pl.when(i == 0)
    def _():
        xbf_ref[...] = x_ref[...].astype(jnp.bfloat16)

    al = a_ref[0, 0]
    c1 = (1.0 - _THETA) * (1.0 - al)
    c2 = (1.0 - _THETA) * al
    row = jax.lax.broadcasted_iota(jnp.int32, (_D, _D), 0)
    col = jax.lax.broadcasted_iota(jnp.int32, (_D, _D), 1)
    eye = (row == col).astype(jnp.float32)
    a_mat = _THETA * w_ref[:_D, :] + c1 * eye
    b_mat = _THETA * w_ref[_D:, :] + c2 * eye
    hb = (
        jnp.dot(rx_ref[...], b_mat[: _D // 2, :], preferred_element_type=jnp.float32)
        + jnp.dot(rl_ref[...], b_mat[_D // 2 :, :], preferred_element_type=jnp.float32)
    )
    x = xbf_ref[...]
    for s, adj_ref in enumerate((adj0_ref, adj1_ref)):
        hi = jnp.dot(
            adj_ref[...].astype(jnp.bfloat16), x, preferred_element_type=jnp.float32
        )
        out_ref[s * _BH : (s + 1) * _BH, :] = (
            jnp.dot(hi, a_mat, preferred_element_type=jnp.float32)
            + hb[s * _BH : (s + 1) * _BH, :]
        )


def kernel(input, adj, Rxyz, Rlamda, alpha, weight, t, l):
    del t, l
    alpha2d = jnp.reshape(alpha.astype(jnp.float32), (1, 1))

    def _adj_spec(s):
        return pl.BlockSpec((_BH, _N), lambda i, s=s: (_NS * i + s, 0))

    out = pl.pallas_call(
        _body,
        grid=(_N // _BM,),
        in_specs=[pl.BlockSpec(memory_space=pltpu.SMEM)]
        + [_adj_spec(s) for s in range(_NS)]
        + [
            pl.BlockSpec((_N, _D), lambda i: (0, 0)),
            pl.BlockSpec((_BM, _D // 2), lambda i: (i, 0)),
            pl.BlockSpec((_BM, _D // 2), lambda i: (i, 0)),
            pl.BlockSpec((2 * _D, _D), lambda i: (0, 0)),
        ],
        out_specs=pl.BlockSpec((_BM, _D), lambda i: (i, 0)),
        out_shape=jax.ShapeDtypeStruct((_N, _D), jnp.float32),
        scratch_shapes=[pltpu.VMEM((_N, _D), jnp.bfloat16)],
        compiler_params=pltpu.CompilerParams(
            dimension_semantics=("arbitrary",),
        ),
    )(alpha2d, adj, adj, input, Rxyz, Rlamda, weight)
    return out
